# baseline (speedup 1.0000x reference)
import functools

import jax
import jax.numpy as jnp
from jax import lax
from jax.experimental import pallas as pl
from jax.experimental.pallas import tpu as pltpu
from jax.experimental.pallas import tpu_sc as plsc

N = 100000
D = 128
C = 128
N_PAD = ((N + C - 1) // C) * C
NCHUNK = N_PAD // C
TAIL = N - (NCHUNK - 1) * C

_info = plsc.get_sparse_core_info()
NC, NS = _info.num_cores, _info.num_subcores
NW = NC * NS
STEPS = (NCHUNK + NW - 1) // NW
NBUF = 2
OUTER = (STEPS + NBUF - 1) // NBUF


def _body(xt, t0, t1, t2, out, i0, i1, rb0, rb1, gs0, gs1, ws0, ws1):
    idxv = [i0, i1]
    rows = [rb0, rb1]
    gsem = [gs0, gs1]
    wsem = [ws0, ws1]
    tabs = [t0, t1, t2]
    wid = lax.axis_index("s") * NC + lax.axis_index("c")

    def outer(i, carry):
        for b in range(NBUF):
            kk = i * NBUF + b
            c = wid + kk * NW
            prev_c = c - NBUF * NW

            @pl.when(jnp.logical_and(kk >= NBUF, prev_c < NCHUNK - 1))
            def _(b=b):
                pltpu.make_async_copy(
                    rows[b], out.at[pl.ds(0, C), :], wsem[b]
                ).wait()

            @pl.when(c < NCHUNK)
            def _(b=b, c=c):
                base = c * C
                pltpu.sync_copy(xt.at[:, pl.ds(base, C)], idxv[b])
                gs = [
                    pltpu.async_copy(tabs[t].at[idxv[b].at[t]],
                                     rows[b].at[:, pl.ds(t * D, D)],
                                     gsem[b])
                    for t in range(3)
                ]
                for g in gs:
                    g.wait()

                @pl.when(c < NCHUNK - 1)
                def _(b=b):
                    pltpu.async_copy(rows[b], out.at[pl.ds(base, C), :],
                                     wsem[b])

                @pl.when(c == NCHUNK - 1)
                def _(b=b):
                    pltpu.sync_copy(rows[b].at[pl.ds(0, TAIL), :],
                                    out.at[pl.ds(base, TAIL), :])

        return carry

    lax.fori_loop(0, OUTER, outer, 0)

    last_c = wid + (STEPS - 1) * NW

    @pl.when(last_c < NCHUNK - 1)
    def _():
        pltpu.make_async_copy(
            rows[0], out.at[pl.ds(0, C), :], wsem[0]
        ).wait()


@jax.jit
def _run(xt, t0, t1, t2):
    mesh = plsc.VectorSubcoreMesh(core_axis_name="c", subcore_axis_name="s")
    f = pl.kernel(
        _body,
        out_type=jax.ShapeDtypeStruct((N, 3 * D), jnp.float32),
        mesh=mesh,
        scratch_types=[
            pltpu.VMEM((3, C), jnp.int32),
            pltpu.VMEM((3, C), jnp.int32),
            pltpu.VMEM((C, 3 * D), jnp.float32),
            pltpu.VMEM((C, 3 * D), jnp.float32),
            pltpu.SemaphoreType.DMA,
            pltpu.SemaphoreType.DMA,
            pltpu.SemaphoreType.DMA,
            pltpu.SemaphoreType.DMA,
        ],
    )
    return f(xt, t0, t1, t2)


def kernel(x, t0, t1, t2):
    xt = jnp.pad(x.astype(jnp.int32).T, ((0, 0), (0, N_PAD - N)))
    return _run(xt, t0, t1, t2)


# speedup vs baseline: 1.9790x; 1.9790x over previous
import functools

import jax
import jax.numpy as jnp
from jax import lax
from jax.experimental import pallas as pl
from jax.experimental.pallas import tpu as pltpu
from jax.experimental.pallas import tpu_sc as plsc

N = 100000
D = 128
C = 128
N_PAD = ((N + C - 1) // C) * C
NCHUNK = N_PAD // C
TAIL = N - (NCHUNK - 1) * C

_info = plsc.get_sparse_core_info()
NC, NS = _info.num_cores, _info.num_subcores
NW = NC * NS
STEPS = (NCHUNK + NW - 1) // NW
NBUF = 2
OUTER = (STEPS + NBUF - 1) // NBUF


def _body(xt, t0, t1, t2, out, i0, i1, rb0, rb1, s0, s1, s2, ws0, ws1):
    idxv = [i0, i1]
    rows = [rb0, rb1]
    wsem = [ws0, ws1]
    tabs = [s0, s1, s2]
    wid = lax.axis_index("s") * NC + lax.axis_index("c")

    pltpu.sync_copy(t0, s0)
    pltpu.sync_copy(t1, s1)
    pltpu.sync_copy(t2, s2)

    def assemble(b):
        def grp_body(g, carry):
            iv0 = idxv[b][0, pl.ds(g * 16, 16)]
            iv1 = idxv[b][1, pl.ds(g * 16, 16)]
            iv2 = idxv[b][2, pl.ds(g * 16, 16)]
            for l in range(16):
                r = g * 16 + l
                j0, j1, j2 = iv0[l], iv1[l], iv2[l]
                for jj in range(D // 16):
                    rows[b][r, pl.ds(jj * 16, 16)] = \
                        s0[j0, pl.ds(jj * 16, 16)]
                for jj in range(D // 16):
                    rows[b][r, pl.ds(D + jj * 16, 16)] = \
                        s1[j1, pl.ds(jj * 16, 16)]
                for jj in range(D // 16):
                    rows[b][r, pl.ds(2 * D + jj * 16, 16)] = \
                        s2[j2, pl.ds(jj * 16, 16)]
            return carry

        lax.fori_loop(0, C // 16, grp_body, 0)

    def outer(i, carry):
        for b in range(NBUF):
            kk = i * NBUF + b
            c = wid + kk * NW
            prev_c = c - NBUF * NW

            @pl.when(jnp.logical_and(kk >= NBUF, prev_c < NCHUNK - 1))
            def _(b=b):
                pltpu.make_async_copy(
                    rows[b], out.at[pl.ds(0, C), :], wsem[b]
                ).wait()

            @pl.when(c < NCHUNK)
            def _(b=b, c=c):
                base = c * C
                pltpu.sync_copy(xt.at[:, pl.ds(base, C)], idxv[b])
                assemble(b)

                @pl.when(c < NCHUNK - 1)
                def _(b=b):
                    pltpu.async_copy(rows[b], out.at[pl.ds(base, C), :],
                                     wsem[b])

                @pl.when(c == NCHUNK - 1)
                def _(b=b):
                    pltpu.sync_copy(rows[b].at[pl.ds(0, TAIL), :],
                                    out.at[pl.ds(base, TAIL), :])

        return carry

    lax.fori_loop(0, OUTER, outer, 0)

    last_c = wid + (STEPS - 1) * NW

    @pl.when(last_c < NCHUNK - 1)
    def _():
        pltpu.make_async_copy(
            rows[0], out.at[pl.ds(0, C), :], wsem[0]
        ).wait()


@jax.jit
def _run(xt, t0, t1, t2):
    mesh = plsc.VectorSubcoreMesh(core_axis_name="c", subcore_axis_name="s")
    f = pl.kernel(
        _body,
        out_type=jax.ShapeDtypeStruct((N, 3 * D), jnp.float32),
        mesh=mesh,
        scratch_types=[
            pltpu.VMEM((3, C), jnp.int32),
            pltpu.VMEM((3, C), jnp.int32),
            pltpu.VMEM((C, 3 * D), jnp.float32),
            pltpu.VMEM((C, 3 * D), jnp.float32),
            pltpu.VMEM((16, D), jnp.float32),
            pltpu.VMEM((32, D), jnp.float32),
            pltpu.VMEM((128, D), jnp.float32),
            pltpu.SemaphoreType.DMA,
            pltpu.SemaphoreType.DMA,
        ],
    )
    return f(xt, t0, t1, t2)


def kernel(x, t0, t1, t2):
    xt = jnp.pad(x.astype(jnp.int32).T, ((0, 0), (0, N_PAD - N)))
    return _run(xt, t0, t1, t2)
